# Initial kernel scaffold; baseline (speedup 1.0000x reference)
import jax
import jax.numpy as jnp
from jax.experimental import pallas as pl


def kernel(hx, idxn, segment_ids, edgefeats, W1, b1, W2, b2, W_ih, W_hh, b_ih, b_hh):
    raise NotImplementedError("write your pallas kernel here")


# speedup vs baseline: 3.1434x; 3.1434x over previous
import functools

import jax
import jax.numpy as jnp
from jax import lax
from jax.experimental import pallas as pl
from jax.experimental.pallas import tpu as pltpu
from jax.experimental.pallas import tpu_sc as plsc

N = 10000
E = 320000
NC = 128
DE = 16
FH = 64
NREP = 2

CHUNK = 128
NCHUNKS = E // CHUNK
NCORES = 2
NSUB = 16
NW = NCORES * NSUB
HALF = N // NCORES
ROWS_PER_SUB = 312
ROWS_LAST = HALF - ROWS_PER_SUB * (NSUB - 1)

E_BLK = 2000
N_BLK = 2000


def _edge_mlp_body(ef_ref, w1_ref, b1_ref, w2_ref, b2_ref, out_ref):
    hmid = jnp.dot(ef_ref[...], w1_ref[...], preferred_element_type=jnp.float32)
    hmid = jnp.maximum(hmid + b1_ref[...], 0.0)
    out_ref[...] = (
        jnp.dot(hmid, w2_ref[...], preferred_element_type=jnp.float32) + b2_ref[...]
    )


def _edge_mlp(edgefeats, W1, b1, W2, b2):
    grid = (E // E_BLK,)
    return pl.pallas_call(
        _edge_mlp_body,
        grid=grid,
        in_specs=[
            pl.BlockSpec((E_BLK, DE), lambda i: (i, 0)),
            pl.BlockSpec((DE, FH), lambda i: (0, 0)),
            pl.BlockSpec((1, FH), lambda i: (0, 0)),
            pl.BlockSpec((FH, NC), lambda i: (0, 0)),
            pl.BlockSpec((1, NC), lambda i: (0, 0)),
        ],
        out_specs=pl.BlockSpec((E_BLK, NC), lambda i: (i, 0)),
        out_shape=jax.ShapeDtypeStruct((E, NC), jnp.float32),
    )(edgefeats, W1, b1, W2, b2)


_SC_MESH = plsc.VectorSubcoreMesh(core_axis_name="c", subcore_axis_name="s")


@functools.partial(
    pl.kernel,
    mesh=_SC_MESH,
    out_type=(
        jax.ShapeDtypeStruct((N, NC), jnp.float32),
        jax.ShapeDtypeStruct((N // N_BLK, NW, N_BLK), jnp.float32),
    ),
    scratch_types=[
        pltpu.VMEM((CHUNK,), jnp.int32),
        pltpu.VMEM((CHUNK,), jnp.int32),
        pltpu.VMEM((CHUNK,), jnp.int32),
        pltpu.VMEM((CHUNK, NC), jnp.float32),
        pltpu.VMEM((CHUNK, NC), jnp.float32),
        pltpu.VMEM((CHUNK, NC), jnp.float32),
        pltpu.VMEM((N // N_BLK, N_BLK), jnp.float32),
        pltpu.VMEM_SHARED((HALF, NC), jnp.float32),
        pltpu.SemaphoreType.DMA,
    ],
    compiler_params=pltpu.CompilerParams(needs_layout_passes=False),
)
def _sc_gather_scatter(h_hbm, idx_hbm, seg_hbm, w_hbm, zeros_hbm, zeros1_hbm,
                       agg_out, deg_out,
                       idx_v, seg_v, loc_v, rows_v, w_v, prod_v, deg_v,
                       agg_sp, sem):
    cid = lax.axis_index("c")
    sid = lax.axis_index("s")
    wid = sid * NCORES + cid
    lo = cid * HALF
    hi = lo + HALF

    @pl.when(sid < NSUB - 1)
    def _():
        pltpu.sync_copy(zeros_hbm.at[pl.ds(0, ROWS_PER_SUB)],
                        agg_sp.at[pl.ds(sid * ROWS_PER_SUB, ROWS_PER_SUB)])

    @pl.when(sid == NSUB - 1)
    def _():
        pltpu.sync_copy(zeros_hbm,
                        agg_sp.at[pl.ds((NSUB - 1) * ROWS_PER_SUB, ROWS_LAST)])

    pltpu.sync_copy(zeros1_hbm, deg_v)
    plsc.subcore_barrier()

    ones16 = jnp.full((16,), 1.0, dtype=jnp.float32)

    nmine = jnp.where(sid < NCHUNKS - (NCHUNKS // NSUB) * NSUB,
                      NCHUNKS // NSUB + 1, NCHUNKS // NSUB)

    def body(t, carry):
        j = sid + t * NSUB
        pltpu.sync_copy(seg_hbm.at[j], seg_v)
        s_first = seg_v[pl.ds(0, 16)][0]
        s_last = seg_v[pl.ds(CHUNK - 16, 16)][15]

        @pl.when((s_last >= lo) & (s_first < hi))
        def _():
            pltpu.sync_copy(idx_hbm.at[j], idx_v)
            gather = pltpu.async_copy(h_hbm.at[idx_v], rows_v, sem)
            pltpu.sync_copy(w_hbm.at[pl.ds(j * CHUNK, CHUNK)], w_v)
            for i in range(CHUNK // 16):
                sl = pl.ds(i * 16, 16)
                seg16 = seg_v[sl]
                inr = (seg16 >= lo) & (seg16 < hi)
                loc_v[sl] = jnp.where(inr, seg16 - lo, -1)
                plsc.addupdate_scatter(
                    deg_v, [seg16 // N_BLK, seg16 % N_BLK], ones16, mask=inr)
            gather.wait()

            def mul_row(r, c2):
                for c in range(NC // 16):
                    csl = pl.ds(c * 16, 16)
                    prod_v[r, csl] = rows_v[r, csl] * w_v[r, csl]
                return c2

            lax.fori_loop(0, CHUNK, mul_row, 0)
            pltpu.sync_copy(
                prod_v,
                agg_sp.at[plsc.Indices(loc_v, ignored_value=-1)],
                add=True)

        return carry

    lax.fori_loop(0, nmine, body, 0)
    plsc.subcore_barrier()

    @pl.when(sid < NSUB - 1)
    def _():
        pltpu.sync_copy(
            agg_sp.at[pl.ds(sid * ROWS_PER_SUB, ROWS_PER_SUB)],
            agg_out.at[pl.ds(lo + sid * ROWS_PER_SUB, ROWS_PER_SUB)])

    @pl.when(sid == NSUB - 1)
    def _():
        pltpu.sync_copy(
            agg_sp.at[pl.ds((NSUB - 1) * ROWS_PER_SUB, ROWS_LAST)],
            agg_out.at[pl.ds(lo + (NSUB - 1) * ROWS_PER_SUB, ROWS_LAST)])

    for k in range(N // N_BLK):
        pltpu.sync_copy(deg_v.at[pl.ds(k, 1)], deg_out.at[pl.ds(k, 1), wid])


def _gru_body(agg_ref, degp_ref, h_ref, wih_ref, whh_ref, bih_ref, bhh_ref,
              out_ref):
    deg = jnp.maximum(jnp.sum(degp_ref[0], axis=0), 1.0)[:, None]
    x = agg_ref[...] / deg
    h = h_ref[...]
    gi = jnp.dot(x, wih_ref[...], preferred_element_type=jnp.float32) + bih_ref[...]
    gh = jnp.dot(h, whh_ref[...], preferred_element_type=jnp.float32) + bhh_ref[...]
    r = jax.nn.sigmoid(gi[:, :NC] + gh[:, :NC])
    z = jax.nn.sigmoid(gi[:, NC:2 * NC] + gh[:, NC:2 * NC])
    n = jnp.tanh(gi[:, 2 * NC:] + r * gh[:, 2 * NC:])
    out_ref[...] = (1.0 - z) * n + z * h


def _gru(agg, deg_parts, h, W_ih, W_hh, b_ih, b_hh):
    grid = (N // N_BLK,)
    return pl.pallas_call(
        _gru_body,
        grid=grid,
        in_specs=[
            pl.BlockSpec((N_BLK, NC), lambda i: (i, 0)),
            pl.BlockSpec((1, NW, N_BLK), lambda i: (i, 0, 0)),
            pl.BlockSpec((N_BLK, NC), lambda i: (i, 0)),
            pl.BlockSpec((NC, 3 * NC), lambda i: (0, 0)),
            pl.BlockSpec((NC, 3 * NC), lambda i: (0, 0)),
            pl.BlockSpec((1, 3 * NC), lambda i: (0, 0)),
            pl.BlockSpec((1, 3 * NC), lambda i: (0, 0)),
        ],
        out_specs=pl.BlockSpec((N_BLK, NC), lambda i: (i, 0)),
        out_shape=jax.ShapeDtypeStruct((N, NC), jnp.float32),
    )(agg, deg_parts, h, W_ih, W_hh, b_ih, b_hh)


def kernel(hx, idxn, segment_ids, edgefeats, W1, b1, W2, b2, W_ih, W_hh, b_ih, b_hh):
    weights = _edge_mlp(edgefeats, W1, b1.reshape(1, FH), W2, b2.reshape(1, NC))
    idx2d = idxn.reshape(NCHUNKS, CHUNK)
    seg2d = segment_ids.reshape(NCHUNKS, CHUNK)
    zeros2d = jnp.zeros((ROWS_LAST, NC), jnp.float32)
    zeros1d = jnp.zeros((N // N_BLK, N_BLK), jnp.float32)
    bih2 = b_ih.reshape(1, 3 * NC)
    bhh2 = b_hh.reshape(1, 3 * NC)
    h = hx
    for _ in range(NREP):
        agg, deg_parts = _sc_gather_scatter(
            h, idx2d, seg2d, weights, zeros2d, zeros1d)
        h = _gru(agg, deg_parts, h, W_ih, W_hh, bih2, bhh2)
    return h
